# baseline (speedup 1.0000x reference)
import functools

import jax
import jax.numpy as jnp
from jax import lax
from jax.experimental import pallas as pl
from jax.experimental.pallas import tpu as pltpu
from jax.experimental.pallas import tpu_sc as plsc

_VOCAB = 100000
_D = 162
_DP = 256
_BATCH = 4096
_SEQ = 50
_B = _BATCH * _SEQ

_NC = 2
_NS = 16
_NW = _NC * _NS
_CHUNK = 128
_PER_W = _B // _NW
_NCHUNK = _PER_W // _CHUNK
_NBUF = 2


def _sc_gather(nid_flat, table_pad):
    mesh = plsc.VectorSubcoreMesh(core_axis_name="c", subcore_axis_name="s")

    @functools.partial(
        pl.kernel,
        out_type=jax.ShapeDtypeStruct((_B, _DP), jnp.float32),
        mesh=mesh,
        scratch_types=[
            *[pltpu.VMEM((_CHUNK,), jnp.int32) for _ in range(_NBUF)],
            *[pltpu.VMEM((_CHUNK, _DP), jnp.float32) for _ in range(_NBUF)],
            *[pltpu.SemaphoreType.DMA for _ in range(2 * _NBUF)],
        ],
        compiler_params=pltpu.CompilerParams(use_tc_tiling_on_sc=True),
    )
    def k(idx_hbm, table_hbm, out_hbm, *rest):
        idxb = rest[:_NBUF]
        bufs = rest[_NBUF : 2 * _NBUF]
        gsem = rest[2 * _NBUF : 3 * _NBUF]
        osem = rest[3 * _NBUF : 4 * _NBUF]
        wid = lax.axis_index("s") * _NC + lax.axis_index("c")
        base = wid * _PER_W

        def gather(c, slot):
            pltpu.sync_copy(
                idx_hbm.at[pl.ds(base + c * _CHUNK, _CHUNK)], idxb[slot]
            )
            pltpu.async_copy(table_hbm.at[idxb[slot]], bufs[slot], gsem[slot])

        def gwait(slot):
            pltpu.make_async_copy(
                table_hbm.at[idxb[slot]], bufs[slot], gsem[slot]
            ).wait()

        def copyout(c, slot):
            pltpu.async_copy(
                bufs[slot],
                out_hbm.at[pl.ds(base + c * _CHUNK, _CHUNK)],
                osem[slot],
            ).wait()

        for b in range(_NBUF):
            gather(b, b)

        @pl.loop(0, _NCHUNK)
        def _(c):
            for b in range(_NBUF):
                @pl.when(c % _NBUF == b)
                def _():
                    gwait(b)
                    copyout(c, b)

                    @pl.when(c + _NBUF < _NCHUNK)
                    def _():
                        gather(c + _NBUF, b)

    return k(nid_flat, table_pad)


def _tc_pad(table):
    rows_blk = 2000
    grid = _VOCAB // rows_blk

    def body(t_ref, o_ref):
        o_ref[:, : _D] = t_ref[...]
        o_ref[:, _D:] = jnp.zeros((rows_blk, _DP - _D), jnp.float32)

    return pl.pallas_call(
        body,
        grid=(grid,),
        in_specs=[pl.BlockSpec((rows_blk, _D), lambda i: (i, 0))],
        out_specs=pl.BlockSpec((rows_blk, _DP), lambda i: (i, 0)),
        out_shape=jax.ShapeDtypeStruct((_VOCAB, _DP), jnp.float32),
    )(table)


def _tc_depad(out_pad):
    b_blk = 32
    rows_blk = b_blk * _SEQ
    grid = _B // rows_blk

    def body(p_ref, o_ref):
        o_ref[...] = p_ref[...].reshape(b_blk, _SEQ, _DP)[:, :, : _D]

    return pl.pallas_call(
        body,
        grid=(grid,),
        in_specs=[pl.BlockSpec((rows_blk, _DP), lambda i: (i, 0))],
        out_specs=pl.BlockSpec((b_blk, _SEQ, _D), lambda i: (i, 0, 0)),
        out_shape=jax.ShapeDtypeStruct((_BATCH, _SEQ, _D), jnp.float32),
    )(out_pad)


def kernel(nid, table):
    table_pad = jnp.pad(table, ((0, 0), (0, _DP - _D)))
    out_pad = _sc_gather(nid.reshape(_B), table_pad)
    return out_pad[:, :_D].reshape(_BATCH, _SEQ, _D)


# speedup vs baseline: 1.4945x; 1.4945x over previous
import functools

import jax
import jax.numpy as jnp
from jax import lax
from jax.experimental import pallas as pl
from jax.experimental.pallas import tpu as pltpu
from jax.experimental.pallas import tpu_sc as plsc

_VOCAB = 100000
_D = 162
_DP = 256
_BATCH = 4096
_SEQ = 50
_B = _BATCH * _SEQ

_NC = 2
_NS = 16
_NW = _NC * _NS
_CHUNK = 128
_PER_W = _B // _NW
_NCHUNK = _PER_W // _CHUNK
_NBUF = 2


def _sc_gather(nid_flat, table_pad):
    mesh = plsc.VectorSubcoreMesh(core_axis_name="c", subcore_axis_name="s")

    @functools.partial(
        pl.kernel,
        out_type=jax.ShapeDtypeStruct((_B, _DP), jnp.float32),
        mesh=mesh,
        scratch_types=[
            *[pltpu.VMEM((_CHUNK,), jnp.int32) for _ in range(_NBUF)],
            *[pltpu.VMEM((_CHUNK, _DP), jnp.float32) for _ in range(_NBUF)],
            *[pltpu.SemaphoreType.DMA for _ in range(2 * _NBUF)],
        ],
        compiler_params=pltpu.CompilerParams(use_tc_tiling_on_sc=True),
    )
    def k(idx_hbm, table_hbm, out_hbm, *rest):
        idxb = rest[:_NBUF]
        bufs = rest[_NBUF : 2 * _NBUF]
        gsem = rest[2 * _NBUF : 3 * _NBUF]
        osem = rest[3 * _NBUF : 4 * _NBUF]
        wid = lax.axis_index("s") * _NC + lax.axis_index("c")
        base = wid * _PER_W

        def gather(c, slot):
            pltpu.sync_copy(
                idx_hbm.at[pl.ds(base + c * _CHUNK, _CHUNK)], idxb[slot]
            )
            pltpu.async_copy(table_hbm.at[idxb[slot]], bufs[slot], gsem[slot])

        def gwait(slot):
            pltpu.make_async_copy(
                table_hbm.at[idxb[slot]], bufs[slot], gsem[slot]
            ).wait()

        def copyout(c, slot):
            pltpu.async_copy(
                bufs[slot],
                out_hbm.at[pl.ds(base + c * _CHUNK, _CHUNK)],
                osem[slot],
            ).wait()

        for b in range(_NBUF):
            gather(b, b)

        @pl.loop(0, _NCHUNK)
        def _(c):
            for b in range(_NBUF):
                @pl.when(c % _NBUF == b)
                def _():
                    gwait(b)
                    copyout(c, b)

                    @pl.when(c + _NBUF < _NCHUNK)
                    def _():
                        gather(c + _NBUF, b)

    return k(nid_flat, table_pad)


def _tc_pad(table):
    rows_blk = 2000
    grid = _VOCAB // rows_blk

    def body(t_ref, o_ref):
        o_ref[:, : _D] = t_ref[...]
        o_ref[:, _D:] = jnp.zeros((rows_blk, _DP - _D), jnp.float32)

    return pl.pallas_call(
        body,
        grid=(grid,),
        in_specs=[pl.BlockSpec((rows_blk, _D), lambda i: (i, 0))],
        out_specs=pl.BlockSpec((rows_blk, _DP), lambda i: (i, 0)),
        out_shape=jax.ShapeDtypeStruct((_VOCAB, _DP), jnp.float32),
    )(table)


def _tc_depad(out_pad):
    b_blk = 32
    rows_blk = b_blk * _SEQ
    grid = _B // rows_blk

    def body(p_ref, o_ref):
        o_ref[...] = p_ref[...].reshape(b_blk, _SEQ, _DP)[:, :, : _D]

    return pl.pallas_call(
        body,
        grid=(grid,),
        in_specs=[pl.BlockSpec((rows_blk, _DP), lambda i: (i, 0))],
        out_specs=pl.BlockSpec((b_blk, _SEQ, _D), lambda i: (i, 0, 0)),
        out_shape=jax.ShapeDtypeStruct((_BATCH, _SEQ, _D), jnp.float32),
    )(out_pad)


def kernel(nid, table):
    out_pad = _sc_gather(nid.reshape(_B), _tc_pad(table))
    return out_pad[:, :_D].reshape(_BATCH, _SEQ, _D)
